# Initial kernel scaffold; baseline (speedup 1.0000x reference)
import jax
import jax.numpy as jnp
from jax.experimental import pallas as pl


def kernel(x, edge_index, batch, W1, b1, g1, be1, W2, b2, g2, be2, W3, b3, g3, be3, Wf, bf):
    raise NotImplementedError("write your pallas kernel here")


# speedup vs baseline: 102.3916x; 102.3916x over previous
import functools

import jax
import jax.numpy as jnp
from jax import lax
from jax.experimental import pallas as pl
from jax.experimental.pallas import tpu as pltpu
from jax.experimental.pallas import tpu_sc as plsc

N = 10000
NP = 10240
E = 320000
NUM_GRAPHS = 64
GP = 128
K = 10
ALPHA = 0.1
EPS = 1e-5

NSUB = 16
L = 16
SEG = NP // NSUB
EPT = E // NSUB


def _mlp_body(x_ref, w1_ref, b1_ref, g1_ref, be1_ref,
              w2_ref, b2_ref, g2_ref, be2_ref,
              w3_ref, b3_ref, g3_ref, be3_ref,
              wf_ref, out_ref):
    dn = (((1,), (1,)), ((), ()))

    def bn_relu(h, g, be):
        m = jnp.mean(h, axis=0, keepdims=True)
        d = h - m
        v = jnp.mean(d * d, axis=0, keepdims=True)
        return jnp.maximum(d * lax.rsqrt(v + EPS) * g + be, 0.0)

    h = lax.dot_general(x_ref[...], w1_ref[...], dn,
                        preferred_element_type=jnp.float32) + b1_ref[...]
    h = bn_relu(h, g1_ref[...], be1_ref[...])
    h = lax.dot_general(h, w2_ref[...], dn,
                        preferred_element_type=jnp.float32) + b2_ref[...]
    h = bn_relu(h, g2_ref[...], be2_ref[...])
    h = lax.dot_general(h, w3_ref[...], dn,
                        preferred_element_type=jnp.float32) + b3_ref[...]
    h = bn_relu(h, g3_ref[...], be3_ref[...])
    out_ref[...] = lax.dot_general(wf_ref[...], h, dn,
                                   preferred_element_type=jnp.float32)


_sc_mesh = plsc.VectorSubcoreMesh(core_axis_name="c", subcore_axis_name="s")


@functools.partial(
    pl.kernel,
    out_type=jax.ShapeDtypeStruct((2 * NUM_GRAPHS,), jnp.float32),
    mesh=_sc_mesh,
    compiler_params=pltpu.CompilerParams(needs_layout_passes=False),
    scratch_types=[
        pltpu.VMEM((EPT,), jnp.int32),
        pltpu.VMEM((EPT,), jnp.int32),
        pltpu.VMEM((NP,), jnp.float32),
        pltpu.VMEM((NP,), jnp.float32),
        pltpu.VMEM((NSUB, SEG), jnp.float32),
        pltpu.VMEM((SEG,), jnp.float32),
        pltpu.VMEM((SEG,), jnp.float32),
        pltpu.VMEM((SEG,), jnp.float32),
        pltpu.VMEM((SEG,), jnp.float32),
        pltpu.VMEM((SEG,), jnp.int32),
        pltpu.VMEM((GP,), jnp.float32),
        pltpu.VMEM((GP,), jnp.float32),
        pltpu.VMEM((NSUB, GP), jnp.float32),
        pltpu.VMEM((NSUB, GP), jnp.float32),
        pltpu.VMEM((GP,), jnp.float32),
        pltpu.VMEM_SHARED((NSUB, NP), jnp.float32),
        pltpu.VMEM_SHARED((NP,), jnp.float32),
        pltpu.VMEM_SHARED((NSUB, GP), jnp.float32),
        pltpu.VMEM_SHARED((NSUB, GP), jnp.float32),
    ],
)
def _appnp_sc(row_hbm, col_hbm, batch_hbm, z0_hbm, out_hbm,
              r_v, c_v, u_full, s_full, part,
              z_seg, z0_seg, dinv_seg, u_seg, batch_seg,
              pooled, counts, pool_all, cnt_all, outbuf,
              sh_part, sh_u, sh_pool, sh_cnt):
    col = lax.axis_index("c")
    sid = lax.axis_index("s")
    seg_base = sid * SEG
    e_base = sid * EPT

    zero16f = jnp.zeros((L,), jnp.float32)
    one16f = jnp.ones((L,), jnp.float32)

    pltpu.sync_copy(row_hbm.at[pl.ds(e_base, EPT)], r_v)
    pltpu.sync_copy(col_hbm.at[pl.ds(e_base, EPT)], c_v)
    pltpu.sync_copy(batch_hbm.at[pl.ds(seg_base, SEG)], batch_seg)
    pltpu.sync_copy(z0_hbm.at[pl.ds(col * NP + seg_base, SEG)], z0_seg)

    def _zero_s(i, carry):
        s_full[pl.ds(i * L, L)] = zero16f
        return carry

    lax.fori_loop(0, NP // L, _zero_s, None)

    def _deg_edges(i, carry):
        idx_c = c_v[pl.ds(i * L, L)]
        plsc.addupdate_scatter(s_full, [idx_c], one16f)
        return carry

    lax.fori_loop(0, EPT // L, _deg_edges, None)

    pltpu.sync_copy(s_full, sh_part.at[sid])
    plsc.subcore_barrier()
    pltpu.sync_copy(sh_part.at[:, pl.ds(seg_base, SEG)], part)

    def _dinv_chunk(w, carry):
        sl = pl.ds(w * L, L)
        acc = part[0, sl]
        for j in range(1, NSUB):
            acc = acc + part[j, sl]
        deg = acc + 1.0
        y = plsc.bitcast(jnp.int32(0x5F3759DF) - (plsc.bitcast(deg, jnp.int32) >> 1),
                         jnp.float32)
        hx = 0.5 * deg
        for _ in range(3):
            y = y * (1.5 - hx * y * y)
        dinv_seg[sl] = y
        z_seg[sl] = z0_seg[sl]
        return carry

    lax.fori_loop(0, SEG // L, _dinv_chunk, None)

    def _round(k, carry):
        def _mk_u(w, c2):
            sl = pl.ds(w * L, L)
            u_seg[sl] = dinv_seg[sl] * z_seg[sl]
            return c2

        lax.fori_loop(0, SEG // L, _mk_u, None)
        pltpu.sync_copy(u_seg, sh_u.at[pl.ds(seg_base, SEG)])
        plsc.subcore_barrier()
        pltpu.sync_copy(sh_u, u_full)

        lax.fori_loop(0, NP // L, _zero_s, None)

        def _edges(i, c2):
            sl = pl.ds(i * L, L)
            idx_r = r_v[sl]
            idx_c = c_v[sl]
            vals = plsc.load_gather(u_full, [idx_r])
            plsc.addupdate_scatter(s_full, [idx_c], vals)
            return c2

        lax.fori_loop(0, EPT // L, _edges, None)

        pltpu.sync_copy(s_full, sh_part.at[sid])
        plsc.subcore_barrier()
        pltpu.sync_copy(sh_part.at[:, pl.ds(seg_base, SEG)], part)

        def _update(w, c2):
            sl = pl.ds(w * L, L)
            acc = part[0, sl]
            for j in range(1, NSUB):
                acc = acc + part[j, sl]
            s_tot = acc + u_seg[sl]
            z_seg[sl] = ((1.0 - ALPHA) * (dinv_seg[sl] * s_tot)
                         + ALPHA * z0_seg[sl])
            return c2

        lax.fori_loop(0, SEG // L, _update, None)
        return carry

    lax.fori_loop(0, K, _round, None)

    def _zero_g(w, carry):
        sl = pl.ds(w * L, L)
        pooled[sl] = zero16f
        counts[sl] = zero16f
        return carry

    lax.fori_loop(0, GP // L, _zero_g, None)

    def _pool(w, carry):
        sl = pl.ds(w * L, L)
        b = batch_seg[sl]
        plsc.addupdate_scatter(pooled, [b], z_seg[sl])
        plsc.addupdate_scatter(counts, [b], one16f)
        return carry

    lax.fori_loop(0, SEG // L, _pool, None)

    pltpu.sync_copy(pooled, sh_pool.at[sid])
    pltpu.sync_copy(counts, sh_cnt.at[sid])
    plsc.subcore_barrier()

    @pl.when(sid == 0)
    def _final():
        pltpu.sync_copy(sh_pool, pool_all)
        pltpu.sync_copy(sh_cnt, cnt_all)

        def _fin(w, carry):
            sl = pl.ds(w * L, L)
            pa = pool_all[0, sl]
            ca = cnt_all[0, sl]
            for j in range(1, NSUB):
                pa = pa + pool_all[j, sl]
                ca = ca + cnt_all[j, sl]
            outbuf[sl] = pa / jnp.maximum(ca, 1.0)
            return carry

        lax.fori_loop(0, GP // L, _fin, None)
        pltpu.sync_copy(outbuf.at[pl.ds(0, NUM_GRAPHS)],
                        out_hbm.at[pl.ds(col * NUM_GRAPHS, NUM_GRAPHS)])


def kernel(x, edge_index, batch, W1, b1, g1, be1, W2, b2, g2, be2,
           W3, b3, g3, be3, Wf, bf):
    z0 = pl.pallas_call(
        _mlp_body,
        out_shape=jax.ShapeDtypeStruct((2, N), jnp.float32),
    )(x, W1, b1.reshape(1, -1), g1.reshape(1, -1), be1.reshape(1, -1),
      W2, b2.reshape(1, -1), g2.reshape(1, -1), be2.reshape(1, -1),
      W3, b3.reshape(1, -1), g3.reshape(1, -1), be3.reshape(1, -1),
      Wf)
    z0p = jnp.pad(z0, ((0, 0), (0, NP - N))).reshape(-1)
    batch_p = jnp.concatenate(
        [batch, jnp.full((NP - N,), NUM_GRAPHS, jnp.int32)])
    out_flat = _appnp_sc(edge_index[0], edge_index[1], batch_p, z0p)
    return out_flat.reshape(2, NUM_GRAPHS).T + bf


# speedup vs baseline: 117.4844x; 1.1474x over previous
import functools

import jax
import jax.numpy as jnp
from jax import lax
from jax.experimental import pallas as pl
from jax.experimental.pallas import tpu as pltpu
from jax.experimental.pallas import tpu_sc as plsc

N = 10000
NP = 10240
E = 320000
NUM_GRAPHS = 64
GP = 128
K = 10
ALPHA = 0.1
EPS = 1e-5

NSUB = 16
L = 16
SEG = NP // NSUB
EPT = E // NSUB


def _mlp_body(x_ref, w1_ref, b1_ref, g1_ref, be1_ref,
              w2_ref, b2_ref, g2_ref, be2_ref,
              w3_ref, b3_ref, g3_ref, be3_ref,
              wf_ref, out_ref):
    dn = (((1,), (1,)), ((), ()))

    def bn_relu(h, g, be):
        m = jnp.mean(h, axis=0, keepdims=True)
        d = h - m
        v = jnp.mean(d * d, axis=0, keepdims=True)
        return jnp.maximum(d * lax.rsqrt(v + EPS) * g + be, 0.0)

    h = lax.dot_general(x_ref[...], w1_ref[...], dn,
                        preferred_element_type=jnp.float32) + b1_ref[...]
    h = bn_relu(h, g1_ref[...], be1_ref[...])
    h = lax.dot_general(h, w2_ref[...], dn,
                        preferred_element_type=jnp.float32) + b2_ref[...]
    h = bn_relu(h, g2_ref[...], be2_ref[...])
    h = lax.dot_general(h, w3_ref[...], dn,
                        preferred_element_type=jnp.float32) + b3_ref[...]
    h = bn_relu(h, g3_ref[...], be3_ref[...])
    out_ref[...] = lax.dot_general(wf_ref[...], h, dn,
                                   preferred_element_type=jnp.float32)


_sc_mesh = plsc.VectorSubcoreMesh(core_axis_name="c", subcore_axis_name="s")


@functools.partial(
    pl.kernel,
    out_type=jax.ShapeDtypeStruct((2 * NUM_GRAPHS,), jnp.float32),
    mesh=_sc_mesh,
    compiler_params=pltpu.CompilerParams(needs_layout_passes=False),
    scratch_types=[
        pltpu.VMEM((EPT,), jnp.int32),
        pltpu.VMEM((EPT,), jnp.int32),
        pltpu.VMEM((NP,), jnp.float32),
        pltpu.VMEM((NP,), jnp.float32),
        pltpu.VMEM((NSUB, SEG), jnp.float32),
        pltpu.VMEM((SEG,), jnp.float32),
        pltpu.VMEM((SEG,), jnp.float32),
        pltpu.VMEM((SEG,), jnp.float32),
        pltpu.VMEM((SEG,), jnp.float32),
        pltpu.VMEM((SEG,), jnp.int32),
        pltpu.VMEM((GP,), jnp.float32),
        pltpu.VMEM((GP,), jnp.float32),
        pltpu.VMEM((NSUB, GP), jnp.float32),
        pltpu.VMEM((NSUB, GP), jnp.float32),
        pltpu.VMEM((GP,), jnp.float32),
        pltpu.VMEM_SHARED((NSUB, NP), jnp.float32),
        pltpu.VMEM_SHARED((NP,), jnp.float32),
        pltpu.VMEM_SHARED((NSUB, GP), jnp.float32),
        pltpu.VMEM_SHARED((NSUB, GP), jnp.float32),
    ],
)
def _appnp_sc(row_hbm, col_hbm, batch_hbm, z0_hbm, out_hbm,
              r_v, c_v, u_full, s_full, part,
              z_seg, z0_seg, dinv_seg, u_seg, batch_seg,
              pooled, counts, pool_all, cnt_all, outbuf,
              sh_part, sh_u, sh_pool, sh_cnt):
    col = lax.axis_index("c")
    sid = lax.axis_index("s")
    seg_base = sid * SEG
    e_base = sid * EPT

    zero16f = jnp.zeros((L,), jnp.float32)
    one16f = jnp.ones((L,), jnp.float32)

    pltpu.sync_copy(row_hbm.at[pl.ds(e_base, EPT)], r_v)
    pltpu.sync_copy(col_hbm.at[pl.ds(e_base, EPT)], c_v)
    pltpu.sync_copy(batch_hbm.at[pl.ds(seg_base, SEG)], batch_seg)
    pltpu.sync_copy(z0_hbm.at[pl.ds(col * NP + seg_base, SEG)], z0_seg)

    def _zero_s(i, carry):
        s_full[pl.ds(i * L, L)] = zero16f
        return carry

    lax.fori_loop(0, NP // L, _zero_s, None, unroll=8)

    def _deg_edges(i, carry):
        idx_c = c_v[pl.ds(i * L, L)]
        plsc.addupdate_scatter(s_full, [idx_c], one16f)
        return carry

    lax.fori_loop(0, EPT // L, _deg_edges, None, unroll=8)

    pltpu.sync_copy(s_full, sh_part.at[sid])
    plsc.subcore_barrier()
    pltpu.sync_copy(sh_part.at[:, pl.ds(seg_base, SEG)], part)

    def _dinv_chunk(w, carry):
        sl = pl.ds(w * L, L)
        acc = part[0, sl]
        for j in range(1, NSUB):
            acc = acc + part[j, sl]
        deg = acc + 1.0
        y = plsc.bitcast(jnp.int32(0x5F3759DF) - (plsc.bitcast(deg, jnp.int32) >> 1),
                         jnp.float32)
        hx = 0.5 * deg
        for _ in range(3):
            y = y * (1.5 - hx * y * y)
        dinv_seg[sl] = y
        z_seg[sl] = z0_seg[sl]
        return carry

    lax.fori_loop(0, SEG // L, _dinv_chunk, None)

    def _round(k, carry):
        def _mk_u(w, c2):
            sl = pl.ds(w * L, L)
            u_seg[sl] = dinv_seg[sl] * z_seg[sl]
            return c2

        lax.fori_loop(0, SEG // L, _mk_u, None, unroll=4)
        pltpu.sync_copy(u_seg, sh_u.at[pl.ds(seg_base, SEG)])
        plsc.subcore_barrier()
        pltpu.sync_copy(sh_u, u_full)

        lax.fori_loop(0, NP // L, _zero_s, None, unroll=8)

        def _edges(i, c2):
            sl = pl.ds(i * L, L)
            idx_r = r_v[sl]
            idx_c = c_v[sl]
            vals = plsc.load_gather(u_full, [idx_r])
            plsc.addupdate_scatter(s_full, [idx_c], vals)
            return c2

        lax.fori_loop(0, EPT // L, _edges, None, unroll=8)

        pltpu.sync_copy(s_full, sh_part.at[sid])
        plsc.subcore_barrier()
        pltpu.sync_copy(sh_part.at[:, pl.ds(seg_base, SEG)], part)

        def _update(w, c2):
            sl = pl.ds(w * L, L)
            acc = part[0, sl]
            for j in range(1, NSUB):
                acc = acc + part[j, sl]
            s_tot = acc + u_seg[sl]
            z_seg[sl] = ((1.0 - ALPHA) * (dinv_seg[sl] * s_tot)
                         + ALPHA * z0_seg[sl])
            return c2

        lax.fori_loop(0, SEG // L, _update, None, unroll=4)
        return carry

    lax.fori_loop(0, K, _round, None)

    def _zero_g(w, carry):
        sl = pl.ds(w * L, L)
        pooled[sl] = zero16f
        counts[sl] = zero16f
        return carry

    lax.fori_loop(0, GP // L, _zero_g, None)

    def _pool(w, carry):
        sl = pl.ds(w * L, L)
        b = batch_seg[sl]
        plsc.addupdate_scatter(pooled, [b], z_seg[sl])
        plsc.addupdate_scatter(counts, [b], one16f)
        return carry

    lax.fori_loop(0, SEG // L, _pool, None)

    pltpu.sync_copy(pooled, sh_pool.at[sid])
    pltpu.sync_copy(counts, sh_cnt.at[sid])
    plsc.subcore_barrier()

    @pl.when(sid == 0)
    def _final():
        pltpu.sync_copy(sh_pool, pool_all)
        pltpu.sync_copy(sh_cnt, cnt_all)

        def _fin(w, carry):
            sl = pl.ds(w * L, L)
            pa = pool_all[0, sl]
            ca = cnt_all[0, sl]
            for j in range(1, NSUB):
                pa = pa + pool_all[j, sl]
                ca = ca + cnt_all[j, sl]
            outbuf[sl] = pa / jnp.maximum(ca, 1.0)
            return carry

        lax.fori_loop(0, GP // L, _fin, None)
        pltpu.sync_copy(outbuf.at[pl.ds(0, NUM_GRAPHS)],
                        out_hbm.at[pl.ds(col * NUM_GRAPHS, NUM_GRAPHS)])


def kernel(x, edge_index, batch, W1, b1, g1, be1, W2, b2, g2, be2,
           W3, b3, g3, be3, Wf, bf):
    z0 = pl.pallas_call(
        _mlp_body,
        out_shape=jax.ShapeDtypeStruct((2, N), jnp.float32),
    )(x, W1, b1.reshape(1, -1), g1.reshape(1, -1), be1.reshape(1, -1),
      W2, b2.reshape(1, -1), g2.reshape(1, -1), be2.reshape(1, -1),
      W3, b3.reshape(1, -1), g3.reshape(1, -1), be3.reshape(1, -1),
      Wf)
    z0p = jnp.pad(z0, ((0, 0), (0, NP - N))).reshape(-1)
    batch_p = jnp.concatenate(
        [batch, jnp.full((NP - N,), NUM_GRAPHS, jnp.int32)])
    out_flat = _appnp_sc(edge_index[0], edge_index[1], batch_p, z0p)
    return out_flat.reshape(2, NUM_GRAPHS).T + bf


# speedup vs baseline: 316.6967x; 2.6956x over previous
import functools

import jax
import jax.numpy as jnp
from jax import lax
from jax.experimental import pallas as pl
from jax.experimental.pallas import tpu as pltpu
from jax.experimental.pallas import tpu_sc as plsc

N = 10000
NP = 10240
E = 320000
NUM_GRAPHS = 64
GP = 128
K = 10
ALPHA = 0.1
EPS = 1e-5

NSUB = 16
L = 16
SEG = NP // NSUB
EPT = E // NSUB


def _mlp_body(x_ref, w1_ref, b1_ref, g1_ref, be1_ref,
              w2_ref, b2_ref, g2_ref, be2_ref,
              w3_ref, b3_ref, g3_ref, be3_ref,
              wf_ref, out_ref):
    dn = (((1,), (1,)), ((), ()))

    def bn_relu(h, g, be):
        m = jnp.mean(h, axis=0, keepdims=True)
        d = h - m
        v = jnp.mean(d * d, axis=0, keepdims=True)
        return jnp.maximum(d * lax.rsqrt(v + EPS) * g + be, 0.0)

    h = lax.dot_general(x_ref[...], w1_ref[...], dn,
                        preferred_element_type=jnp.float32) + b1_ref[...]
    h = bn_relu(h, g1_ref[...], be1_ref[...])
    h = lax.dot_general(h, w2_ref[...], dn,
                        preferred_element_type=jnp.float32) + b2_ref[...]
    h = bn_relu(h, g2_ref[...], be2_ref[...])
    h = lax.dot_general(h, w3_ref[...], dn,
                        preferred_element_type=jnp.float32) + b3_ref[...]
    h = bn_relu(h, g3_ref[...], be3_ref[...])
    out_ref[...] = lax.dot_general(wf_ref[...], h, dn,
                                   preferred_element_type=jnp.float32)


_sc_mesh = plsc.VectorSubcoreMesh(core_axis_name="c", subcore_axis_name="s")


@functools.partial(
    pl.kernel,
    out_type=jax.ShapeDtypeStruct((2 * NUM_GRAPHS,), jnp.float32),
    mesh=_sc_mesh,
    compiler_params=pltpu.CompilerParams(needs_layout_passes=False),
    scratch_types=[
        pltpu.VMEM((EPT,), jnp.int32),
        pltpu.VMEM((EPT,), jnp.int32),
        pltpu.VMEM((NP,), jnp.float32),
        pltpu.VMEM((NP,), jnp.float32),
        pltpu.VMEM((NSUB, SEG), jnp.float32),
        pltpu.VMEM((SEG,), jnp.float32),
        pltpu.VMEM((SEG,), jnp.float32),
        pltpu.VMEM((SEG,), jnp.float32),
        pltpu.VMEM((SEG,), jnp.float32),
        pltpu.VMEM((SEG,), jnp.int32),
        pltpu.VMEM((GP,), jnp.float32),
        pltpu.VMEM((GP,), jnp.float32),
        pltpu.VMEM((NSUB, GP), jnp.float32),
        pltpu.VMEM((NSUB, GP), jnp.float32),
        pltpu.VMEM((GP,), jnp.float32),
        pltpu.VMEM_SHARED((NSUB, NP), jnp.float32),
        pltpu.VMEM_SHARED((NP,), jnp.float32),
        pltpu.VMEM_SHARED((NSUB, GP), jnp.float32),
        pltpu.VMEM_SHARED((NSUB, GP), jnp.float32),
    ],
)
def _appnp_sc(row_hbm, col_hbm, batch_hbm, z0_hbm, out_hbm,
              r_v, c_v, u_full, s_full, part,
              z_seg, z0_seg, dinv_seg, u_seg, batch_seg,
              pooled, counts, pool_all, cnt_all, outbuf,
              sh_part, sh_u, sh_pool, sh_cnt):
    col = lax.axis_index("c")
    sid = lax.axis_index("s")
    seg_base = sid * SEG
    e_base = sid * EPT

    zero16f = jnp.zeros((L,), jnp.float32)
    one16f = jnp.ones((L,), jnp.float32)

    pltpu.sync_copy(row_hbm.at[pl.ds(e_base, EPT)], r_v)
    pltpu.sync_copy(col_hbm.at[pl.ds(e_base, EPT)], c_v)
    pltpu.sync_copy(batch_hbm.at[pl.ds(seg_base, SEG)], batch_seg)
    pltpu.sync_copy(z0_hbm.at[pl.ds(col * NP + seg_base, SEG)], z0_seg)

    def _zero_s_loop():
        @functools.partial(plsc.parallel_loop, 0, NP // L, unroll=8)
        def _zs(i):
            s_full[pl.ds(i * L, L)] = zero16f

    _zero_s_loop()

    @functools.partial(plsc.parallel_loop, 0, EPT // L, unroll=8)
    def _deg_edges(i):
        idx_c = c_v[pl.ds(i * L, L)]
        plsc.addupdate_scatter(s_full, [idx_c], one16f)

    pltpu.sync_copy(s_full, sh_part.at[sid])
    plsc.subcore_barrier()
    pltpu.sync_copy(sh_part.at[:, pl.ds(seg_base, SEG)], part)

    def _dinv_chunk(w, carry):
        sl = pl.ds(w * L, L)
        acc = part[0, sl]
        for j in range(1, NSUB):
            acc = acc + part[j, sl]
        deg = acc + 1.0
        y = plsc.bitcast(jnp.int32(0x5F3759DF) - (plsc.bitcast(deg, jnp.int32) >> 1),
                         jnp.float32)
        hx = 0.5 * deg
        for _ in range(3):
            y = y * (1.5 - hx * y * y)
        dinv_seg[sl] = y
        z_seg[sl] = z0_seg[sl]
        return carry

    lax.fori_loop(0, SEG // L, _dinv_chunk, None)

    def _round(k, carry):
        @functools.partial(plsc.parallel_loop, 0, SEG // L, unroll=4)
        def _mk_u(w):
            sl = pl.ds(w * L, L)
            u_seg[sl] = dinv_seg[sl] * z_seg[sl]

        pltpu.sync_copy(u_seg, sh_u.at[pl.ds(seg_base, SEG)])
        plsc.subcore_barrier()
        pltpu.sync_copy(sh_u, u_full)

        _zero_s_loop()

        @functools.partial(plsc.parallel_loop, 0, EPT // L, unroll=8)
        def _edges(i):
            sl = pl.ds(i * L, L)
            idx_r = r_v[sl]
            idx_c = c_v[sl]
            vals = plsc.load_gather(u_full, [idx_r])
            plsc.addupdate_scatter(s_full, [idx_c], vals)

        pltpu.sync_copy(s_full, sh_part.at[sid])
        plsc.subcore_barrier()
        pltpu.sync_copy(sh_part.at[:, pl.ds(seg_base, SEG)], part)

        @functools.partial(plsc.parallel_loop, 0, SEG // L, unroll=4)
        def _update(w):
            sl = pl.ds(w * L, L)
            acc = part[0, sl]
            for j in range(1, NSUB):
                acc = acc + part[j, sl]
            s_tot = acc + u_seg[sl]
            z_seg[sl] = ((1.0 - ALPHA) * (dinv_seg[sl] * s_tot)
                         + ALPHA * z0_seg[sl])

        return carry

    lax.fori_loop(0, K, _round, None)

    def _zero_g(w, carry):
        sl = pl.ds(w * L, L)
        pooled[sl] = zero16f
        counts[sl] = zero16f
        return carry

    lax.fori_loop(0, GP // L, _zero_g, None)

    def _pool(w, carry):
        sl = pl.ds(w * L, L)
        b = batch_seg[sl]
        plsc.addupdate_scatter(pooled, [b], z_seg[sl])
        plsc.addupdate_scatter(counts, [b], one16f)
        return carry

    lax.fori_loop(0, SEG // L, _pool, None)

    pltpu.sync_copy(pooled, sh_pool.at[sid])
    pltpu.sync_copy(counts, sh_cnt.at[sid])
    plsc.subcore_barrier()

    @pl.when(sid == 0)
    def _final():
        pltpu.sync_copy(sh_pool, pool_all)
        pltpu.sync_copy(sh_cnt, cnt_all)

        def _fin(w, carry):
            sl = pl.ds(w * L, L)
            pa = pool_all[0, sl]
            ca = cnt_all[0, sl]
            for j in range(1, NSUB):
                pa = pa + pool_all[j, sl]
                ca = ca + cnt_all[j, sl]
            outbuf[sl] = pa / jnp.maximum(ca, 1.0)
            return carry

        lax.fori_loop(0, GP // L, _fin, None)
        pltpu.sync_copy(outbuf.at[pl.ds(0, NUM_GRAPHS)],
                        out_hbm.at[pl.ds(col * NUM_GRAPHS, NUM_GRAPHS)])


def kernel(x, edge_index, batch, W1, b1, g1, be1, W2, b2, g2, be2,
           W3, b3, g3, be3, Wf, bf):
    z0 = pl.pallas_call(
        _mlp_body,
        out_shape=jax.ShapeDtypeStruct((2, N), jnp.float32),
    )(x, W1, b1.reshape(1, -1), g1.reshape(1, -1), be1.reshape(1, -1),
      W2, b2.reshape(1, -1), g2.reshape(1, -1), be2.reshape(1, -1),
      W3, b3.reshape(1, -1), g3.reshape(1, -1), be3.reshape(1, -1),
      Wf)
    z0p = jnp.pad(z0, ((0, 0), (0, NP - N))).reshape(-1)
    batch_p = jnp.concatenate(
        [batch, jnp.full((NP - N,), NUM_GRAPHS, jnp.int32)])
    out_flat = _appnp_sc(edge_index[0], edge_index[1], batch_p, z0p)
    return out_flat.reshape(2, NUM_GRAPHS).T + bf
